# baseline (speedup 1.0000x reference)
import functools
import jax
import jax.numpy as jnp
from jax import lax
from jax.experimental import pallas as pl
from jax.experimental.pallas import tpu as pltpu
from jax.experimental.pallas import tpu_sc as plsc

EMBED = 128
BATCH = 16384
HALF = BATCH // 2

_info = plsc.get_sparse_core_info()
_NC, _NS = _info.num_cores, _info.num_subcores
_NW = _NC * _NS
_CHUNK = 128
_B_PER_W = HALF // _NW
_NCHUNK = _B_PER_W // _CHUNK
_NCHUNK_TOT = 2 * _NCHUNK
_NBUF = _NCHUNK_TOT


@functools.partial(
    pl.kernel,
    out_type=[
        jax.ShapeDtypeStruct((HALF, EMBED), jnp.float32),
        jax.ShapeDtypeStruct((HALF, EMBED), jnp.float32),
    ],
    scratch_types=(
        [pltpu.VMEM((_NCHUNK_TOT, _CHUNK), jnp.int32)]
        + [pltpu.VMEM((_CHUNK, EMBED), jnp.float32) for _ in range(_NBUF)]
        + [pltpu.SemaphoreType.DMA for _ in range(2 * _NBUF)]
    ),
    mesh=plsc.VectorSubcoreMesh(core_axis_name="c", subcore_axis_name="s"),
)
def _sc_gather(mana_idx, type_idx, mana_tab, type_tab, out1, out2,
               idx_v, *bufs_and_sems):
    bufs = bufs_and_sems[:_NBUF]
    gsem = bufs_and_sems[_NBUF:2 * _NBUF]
    ssem = bufs_and_sems[2 * _NBUF:]
    wid = lax.axis_index("s") * _NC + lax.axis_index("c")
    base = wid * _B_PER_W

    pltpu.sync_copy(mana_idx.at[pl.ds(wid * _NCHUNK, _NCHUNK)],
                    idx_v.at[pl.ds(0, _NCHUNK)])
    pltpu.sync_copy(type_idx.at[pl.ds(wid * _NCHUNK, _NCHUNK)],
                    idx_v.at[pl.ds(_NCHUNK, _NCHUNK)])

    def tab_of(c):
        return mana_tab if c < _NCHUNK else type_tab

    def out_of(c):
        return out1 if c < _NCHUNK else out2

    def off_of(c):
        return base + (c % _NCHUNK) * _CHUNK

    gd = [None] * _NCHUNK_TOT
    sd = [None] * _NCHUNK_TOT
    for c in range(_NCHUNK_TOT):
        gd[c] = pltpu.async_copy(tab_of(c).at[idx_v.at[c]], bufs[c], gsem[c])
    for c in range(_NCHUNK_TOT):
        gd[c].wait()
        sd[c] = pltpu.async_copy(bufs[c], out_of(c).at[pl.ds(off_of(c), _CHUNK)],
                                 ssem[c])
    for c in range(_NCHUNK_TOT):
        sd[c].wait()


_TM = 2048
_HBLOCKS = HALF // _TM


def _mm_body(g1_ref, g2_ref, w_ref, b_ref, o_ref):
    w1 = w_ref[:EMBED, :]
    w2 = w_ref[EMBED:, :]
    acc = jnp.dot(g1_ref[...], w1, preferred_element_type=jnp.float32)
    acc += jnp.dot(g2_ref[...], w2, preferred_element_type=jnp.float32)
    o_ref[...] = acc + b_ref[...]


def _mm_body_alias(g1_ref, g2_ref, w_ref, b_ref, prev_ref, o_ref):
    del prev_ref
    _mm_body(g1_ref, g2_ref, w_ref, b_ref, o_ref)


def _tc_matmul_h0(g1, g2, W, b2d):
    return pl.pallas_call(
        _mm_body,
        grid=(_HBLOCKS,),
        in_specs=[
            pl.BlockSpec((_TM, EMBED), lambda i: (i, 0)),
            pl.BlockSpec((_TM, EMBED), lambda i: (i, 0)),
            pl.BlockSpec((2 * EMBED, EMBED), lambda i: (0, 0)),
            pl.BlockSpec((1, EMBED), lambda i: (0, 0)),
        ],
        out_specs=pl.BlockSpec((_TM, EMBED), lambda i: (i, 0)),
        out_shape=jax.ShapeDtypeStruct((BATCH, EMBED), jnp.float32),
    )(g1, g2, W, b2d)


def _tc_matmul_h1(g1, g2, W, b2d, prev):
    return pl.pallas_call(
        _mm_body_alias,
        grid=(_HBLOCKS,),
        in_specs=[
            pl.BlockSpec((_TM, EMBED), lambda i: (i, 0)),
            pl.BlockSpec((_TM, EMBED), lambda i: (i, 0)),
            pl.BlockSpec((2 * EMBED, EMBED), lambda i: (0, 0)),
            pl.BlockSpec((1, EMBED), lambda i: (0, 0)),
            pl.BlockSpec(memory_space=pl.ANY),
        ],
        out_specs=pl.BlockSpec((_TM, EMBED), lambda i: (i + _HBLOCKS, 0)),
        out_shape=jax.ShapeDtypeStruct((BATCH, EMBED), jnp.float32),
        input_output_aliases={4: 0},
    )(g1, g2, W, b2d, prev)


@jax.jit
def kernel(mana_token, type_token, mana_table, type_table, W, b):
    mt = mana_token.astype(jnp.int32).reshape(BATCH // _CHUNK, _CHUNK)
    tt = type_token.astype(jnp.int32).reshape(BATCH // _CHUNK, _CHUNK)
    nrow_h = HALF // _CHUNK
    b2d = b.reshape(1, EMBED)

    g1h0, g2h0 = _sc_gather(mt[:nrow_h], tt[:nrow_h], mana_table, type_table)
    g1h1, g2h1 = _sc_gather(mt[nrow_h:], tt[nrow_h:], mana_table, type_table)
    o = _tc_matmul_h0(g1h0, g2h0, W, b2d)
    return _tc_matmul_h1(g1h1, g2h1, W, b2d, o)


# speedup vs baseline: 1.0390x; 1.0390x over previous
import functools
import jax
import jax.numpy as jnp
from jax import lax
from jax.experimental import pallas as pl
from jax.experimental.pallas import tpu as pltpu
from jax.experimental.pallas import tpu_sc as plsc

EMBED = 128
BATCH = 16384
HALF = BATCH // 2

_info = plsc.get_sparse_core_info()
_NC, _NS = _info.num_cores, _info.num_subcores
_NW = _NC * _NS
_CHUNK = 128
_B_PER_W = HALF // _NW
_NCHUNK = _B_PER_W // _CHUNK
_NCHUNK_TOT = 2 * _NCHUNK
_NBUF = _NCHUNK_TOT


@functools.partial(
    pl.kernel,
    out_type=[
        jax.ShapeDtypeStruct((HALF, EMBED), jnp.float32),
        jax.ShapeDtypeStruct((HALF, EMBED), jnp.float32),
    ],
    scratch_types=(
        [pltpu.VMEM((_NCHUNK_TOT, _CHUNK), jnp.int32)]
        + [pltpu.VMEM((_CHUNK, EMBED), jnp.float32) for _ in range(_NBUF)]
        + [pltpu.SemaphoreType.DMA for _ in range(2 * _NBUF)]
    ),
    mesh=plsc.VectorSubcoreMesh(core_axis_name="c", subcore_axis_name="s"),
)
def _sc_gather(mana_idx, type_idx, mana_tab, type_tab, out1, out2,
               idx_v, *bufs_and_sems):
    bufs = bufs_and_sems[:_NBUF]
    gsem = bufs_and_sems[_NBUF:2 * _NBUF]
    ssem = bufs_and_sems[2 * _NBUF:]
    wid = lax.axis_index("s") * _NC + lax.axis_index("c")
    base = wid * _B_PER_W

    pltpu.sync_copy(mana_idx.at[pl.ds(wid * _NCHUNK, _NCHUNK)],
                    idx_v.at[pl.ds(0, _NCHUNK)])
    pltpu.sync_copy(type_idx.at[pl.ds(wid * _NCHUNK, _NCHUNK)],
                    idx_v.at[pl.ds(_NCHUNK, _NCHUNK)])

    def tab_of(c):
        return mana_tab if c < _NCHUNK else type_tab

    def out_of(c):
        return out1 if c < _NCHUNK else out2

    def off_of(c):
        return base + (c % _NCHUNK) * _CHUNK

    gd = [None] * _NCHUNK_TOT
    sd = [None] * _NCHUNK_TOT
    for c in range(_NCHUNK_TOT):
        gd[c] = pltpu.async_copy(tab_of(c).at[idx_v.at[c]], bufs[c], gsem[c])
    for c in range(_NCHUNK_TOT):
        gd[c].wait()
        sd[c] = pltpu.async_copy(bufs[c], out_of(c).at[pl.ds(off_of(c), _CHUNK)],
                                 ssem[c])
    for c in range(_NCHUNK_TOT):
        sd[c].wait()


_TM = 4096
_HBLOCKS = HALF // _TM


def _mm_body(g1_ref, g2_ref, w_ref, b_ref, o_ref):
    w1 = w_ref[:EMBED, :]
    w2 = w_ref[EMBED:, :]
    acc = jnp.dot(g1_ref[...], w1, preferred_element_type=jnp.float32)
    acc += jnp.dot(g2_ref[...], w2, preferred_element_type=jnp.float32)
    o_ref[...] = acc + b_ref[...]


def _mm_body_alias(g1_ref, g2_ref, w_ref, b_ref, prev_ref, o_ref):
    del prev_ref
    _mm_body(g1_ref, g2_ref, w_ref, b_ref, o_ref)


def _tc_matmul_h0(g1, g2, W, b2d):
    return pl.pallas_call(
        _mm_body,
        grid=(_HBLOCKS,),
        in_specs=[
            pl.BlockSpec((_TM, EMBED), lambda i: (i, 0)),
            pl.BlockSpec((_TM, EMBED), lambda i: (i, 0)),
            pl.BlockSpec((2 * EMBED, EMBED), lambda i: (0, 0)),
            pl.BlockSpec((1, EMBED), lambda i: (0, 0)),
        ],
        out_specs=pl.BlockSpec((_TM, EMBED), lambda i: (i, 0)),
        out_shape=jax.ShapeDtypeStruct((BATCH, EMBED), jnp.float32),
    )(g1, g2, W, b2d)


def _tc_matmul_h1(g1, g2, W, b2d, prev):
    return pl.pallas_call(
        _mm_body_alias,
        grid=(_HBLOCKS,),
        in_specs=[
            pl.BlockSpec((_TM, EMBED), lambda i: (i, 0)),
            pl.BlockSpec((_TM, EMBED), lambda i: (i, 0)),
            pl.BlockSpec((2 * EMBED, EMBED), lambda i: (0, 0)),
            pl.BlockSpec((1, EMBED), lambda i: (0, 0)),
            pl.BlockSpec(memory_space=pl.ANY),
        ],
        out_specs=pl.BlockSpec((_TM, EMBED), lambda i: (i + _HBLOCKS, 0)),
        out_shape=jax.ShapeDtypeStruct((BATCH, EMBED), jnp.float32),
        input_output_aliases={4: 0},
    )(g1, g2, W, b2d, prev)


@jax.jit
def kernel(mana_token, type_token, mana_table, type_table, W, b):
    mt = mana_token.astype(jnp.int32).reshape(BATCH // _CHUNK, _CHUNK)
    tt = type_token.astype(jnp.int32).reshape(BATCH // _CHUNK, _CHUNK)
    nrow_h = HALF // _CHUNK
    b2d = b.reshape(1, EMBED)

    g1h0, g2h0 = _sc_gather(mt[:nrow_h], tt[:nrow_h], mana_table, type_table)
    g1h1, g2h1 = _sc_gather(mt[nrow_h:], tt[nrow_h:], mana_table, type_table)
    o = _tc_matmul_h0(g1h0, g2h0, W, b2d)
    return _tc_matmul_h1(g1h1, g2h1, W, b2d, o)
